# baseline (speedup 1.0000x reference)
import jax
import jax.numpy as jnp
from jax import lax
from jax.experimental import pallas as pl
from jax.experimental.pallas import tpu as pltpu
from jax.experimental.pallas import tpu_sc as plsc
from functools import partial

N = 100000
D = 128
G = 1024
H1 = 256
H2 = 128

NC = 2
NS = 16
NW = NC * NS
BLK = 128
NBP = 32
NBUF = 4
NBLOCKS = 800
LASTFULL = N // BLK - 1
NREST = N - (LASTFULL + 1) * BLK
GROWS = G // NS

T = 49152
TBLK = T // BLK
NB = (NBLOCKS - TBLK) // NW
OWNER = (LASTFULL + 1 - TBLK) // NB
OWNIDX = LASTFULL + 1 - (TBLK + OWNER * NB)
CHUNK = 2048
TSTEPS = T // CHUNK
W = 256


def _sc_pool(x_hbm, idsp_hbm, out_hbm, rows_v, idx_v, acc_sh,
             d0, d1, d2, d3, s0, s1):
    c = lax.axis_index("c")
    s = lax.axis_index("s")
    w = c * NS + s

    dsems = (d0, d1, d2, d3)
    ssems = (s0, s1)

    zz = jnp.zeros((16,), jnp.float32)

    @pl.loop(0, GROWS)
    def _(r):
        @pl.loop(0, D, step=16)
        def _(j):
            rows_v[0, r, pl.ds(j, 16)] = zz

    pltpu.sync_copy(rows_v.at[0, pl.ds(0, GROWS)],
                    acc_sh.at[pl.ds(s * GROWS, GROWS)])
    plsc.subcore_barrier()

    pltpu.sync_copy(idsp_hbm.at[pl.ds(w * NBP, NBP)], idx_v)

    base = TBLK + w * NB

    def dma_issue(j):
        pltpu.async_copy(x_hbm.at[pl.ds((base + j) * BLK, BLK)],
                         rows_v.at[j % NBUF], dsems[j % NBUF])

    def dma_wait(j):
        pltpu.make_async_copy(x_hbm.at[pl.ds(0, BLK)],
                              rows_v.at[j % NBUF], dsems[j % NBUF]).wait()

    def sc_issue(j):
        pltpu.async_copy(rows_v.at[j % NBUF],
                         acc_sh.at[idx_v.at[j]], ssems[j % 2], add=True)

    def sc_wait(j):
        pltpu.make_async_copy(rows_v.at[j % NBUF],
                              acc_sh.at[idx_v.at[j]], ssems[j % 2]).wait()

    @pl.when(base <= LASTFULL)
    def _():
        dma_issue(0)

    @pl.when(base + 1 <= LASTFULL)
    def _():
        dma_issue(1)

    for i in range(NB):
        @pl.when(base + i <= LASTFULL)
        def _(i=i):
            dma_wait(i)
            sc_issue(i)
        if i >= 2:
            @pl.when(base + i - 2 <= LASTFULL)
            def _(i=i):
                sc_wait(i - 2)
        if i + 2 < NB:
            @pl.when(base + i + 2 <= LASTFULL)
            def _(i=i):
                dma_issue(i + 2)
    for j in (NB - 2, NB - 1):
        @pl.when(base + j <= LASTFULL)
        def _(j=j):
            sc_wait(j)

    @pl.when(w == OWNER)
    def _():
        @pl.loop(NREST, BLK)
        def _(r):
            @pl.loop(0, D, step=16)
            def _(j):
                rows_v[0, r, pl.ds(j, 16)] = zz

        pltpu.sync_copy(x_hbm.at[pl.ds((LASTFULL + 1) * BLK, NREST)],
                        rows_v.at[0, pl.ds(0, NREST)])
        pltpu.sync_copy(rows_v.at[0], acc_sh.at[idx_v.at[OWNIDX]], add=True)

    plsc.subcore_barrier()
    pltpu.sync_copy(acc_sh.at[pl.ds(s * GROWS, GROWS)],
                    out_hbm.at[c, pl.ds(s * GROWS, GROWS)])


def _tc_pool_kernel(x_ref, ids_ref, out_ref, acc_ref):
    i = pl.program_id(0)

    @pl.when(i == 0)
    def _():
        acc_ref[...] = jnp.zeros_like(acc_ref)

    ids = ids_ref[0, 0, :]
    x = x_ref[...].astype(jnp.bfloat16)

    base = ids_ref[0, 0, 0]
    win_iota = jax.lax.broadcasted_iota(jnp.int32, (W, CHUNK), 0) + base
    onehot_w = (win_iota == ids[None, :]).astype(jnp.bfloat16)
    acc_ref[pl.ds(base, W), :] += jnp.dot(
        onehot_w, x, preferred_element_type=jnp.float32)

    @pl.when(ids[CHUNK - 1] >= base + W)
    def _():
        seg_iota = jax.lax.broadcasted_iota(jnp.int32, (G, CHUNK), 0)
        onehot_f = ((seg_iota == ids[None, :])
                    & (ids[None, :] >= base + W)).astype(jnp.bfloat16)
        acc_ref[pl.ds(0, G), :] += jnp.dot(
            onehot_f, x, preferred_element_type=jnp.float32)

    @pl.when(i == TSTEPS - 1)
    def _():
        out_ref[...] = acc_ref[pl.ds(0, G), :]


def _mlp_kernel(p_ref, t_ref, w1_ref, b1_ref, w2_ref, b2_ref, w3_ref,
                b3_ref, out_ref):
    g = p_ref[0] + p_ref[1] + t_ref[...]
    h = jnp.maximum(
        jnp.dot(g, w1_ref[...], preferred_element_type=jnp.float32)
        + b1_ref[...], 0.0)
    h = jnp.maximum(
        jnp.dot(h, w2_ref[...], preferred_element_type=jnp.float32)
        + b2_ref[...], 0.0)
    out_ref[...] = (
        jnp.dot(h, w3_ref[...], preferred_element_type=jnp.float32)
        + b3_ref[...])


@jax.jit
def kernel(atom_feat, batch, W1, b1, W2, b2, W3, b3):
    ids = batch.astype(jnp.int32)
    idsp = jnp.pad(
        jnp.pad(ids[T:], (0, NBLOCKS * BLK - N)).reshape(NW, NB, BLK),
        ((0, 0), (0, NBP - NB), (0, 0))).reshape(NW * NBP, BLK)

    mesh = plsc.VectorSubcoreMesh(core_axis_name="c", subcore_axis_name="s")
    sc_pool = partial(
        pl.kernel,
        mesh=mesh,
        out_type=jax.ShapeDtypeStruct((NC, G, D), jnp.float32),
        scratch_types=[
            pltpu.VMEM((NBUF, BLK, D), jnp.float32),
            pltpu.VMEM((NBP, BLK), jnp.int32),
            pltpu.VMEM_SHARED((G, D), jnp.float32),
            pltpu.SemaphoreType.DMA,
            pltpu.SemaphoreType.DMA,
            pltpu.SemaphoreType.DMA,
            pltpu.SemaphoreType.DMA,
            pltpu.SemaphoreType.DMA,
            pltpu.SemaphoreType.DMA,
        ],
    )(_sc_pool)
    partials = sc_pool(atom_feat, idsp)

    ids3 = ids[:T].reshape(TSTEPS, 1, CHUNK)
    tc_part = pl.pallas_call(
        _tc_pool_kernel,
        grid=(TSTEPS,),
        in_specs=[
            pl.BlockSpec((CHUNK, D), lambda i: (i, 0)),
            pl.BlockSpec((1, 1, CHUNK), lambda i: (i, 0, 0)),
        ],
        out_specs=pl.BlockSpec((G, D), lambda i: (0, 0)),
        out_shape=jax.ShapeDtypeStruct((G, D), jnp.float32),
        scratch_shapes=[pltpu.VMEM((G + W, D), jnp.float32)],
        compiler_params=pltpu.CompilerParams(
            dimension_semantics=("arbitrary",)),
    )(atom_feat, ids3)

    out = pl.pallas_call(
        _mlp_kernel,
        out_shape=jax.ShapeDtypeStruct((G, 1), jnp.float32),
    )(partials, tc_part, W1, b1.reshape(1, H1), W2, b2.reshape(1, H2),
      W3, b3.reshape(1, 1))
    return out


# speedup vs baseline: 1.0725x; 1.0725x over previous
import jax
import jax.numpy as jnp
from jax import lax
from jax.experimental import pallas as pl
from jax.experimental.pallas import tpu as pltpu
from jax.experimental.pallas import tpu_sc as plsc
from functools import partial

N = 100000
D = 128
G = 1024
H1 = 256
H2 = 128

NC = 2
NS = 16
NW = NC * NS
BLK = 128
NBP = 32
NBUF = 4
NBLOCKS = 800
LASTFULL = N // BLK - 1
NREST = N - (LASTFULL + 1) * BLK
GROWS = G // NS

T = 40960
TBLK = T // BLK
NB = (NBLOCKS - TBLK) // NW
OWNER = (LASTFULL + 1 - TBLK) // NB
OWNIDX = LASTFULL + 1 - (TBLK + OWNER * NB)
CHUNK = 2048
TSTEPS = T // CHUNK
W = 256


def _sc_pool(x_hbm, idsp_hbm, out_hbm, rows_v, idx_v, acc_sh,
             d0, d1, d2, d3, s0, s1):
    c = lax.axis_index("c")
    s = lax.axis_index("s")
    w = c * NS + s

    dsems = (d0, d1, d2, d3)
    ssems = (s0, s1)

    zz = jnp.zeros((16,), jnp.float32)

    @pl.loop(0, GROWS)
    def _(r):
        @pl.loop(0, D, step=16)
        def _(j):
            rows_v[0, r, pl.ds(j, 16)] = zz

    pltpu.sync_copy(rows_v.at[0, pl.ds(0, GROWS)],
                    acc_sh.at[pl.ds(s * GROWS, GROWS)])
    plsc.subcore_barrier()

    pltpu.sync_copy(idsp_hbm.at[pl.ds(w * NBP, NBP)], idx_v)

    base = TBLK + w * NB

    def dma_issue(j):
        pltpu.async_copy(x_hbm.at[pl.ds((base + j) * BLK, BLK)],
                         rows_v.at[j % NBUF], dsems[j % NBUF])

    def dma_wait(j):
        pltpu.make_async_copy(x_hbm.at[pl.ds(0, BLK)],
                              rows_v.at[j % NBUF], dsems[j % NBUF]).wait()

    def sc_issue(j):
        pltpu.async_copy(rows_v.at[j % NBUF],
                         acc_sh.at[idx_v.at[j]], ssems[j % 2], add=True)

    def sc_wait(j):
        pltpu.make_async_copy(rows_v.at[j % NBUF],
                              acc_sh.at[idx_v.at[j]], ssems[j % 2]).wait()

    @pl.when(base <= LASTFULL)
    def _():
        dma_issue(0)

    @pl.when(base + 1 <= LASTFULL)
    def _():
        dma_issue(1)

    for i in range(NB):
        @pl.when(base + i <= LASTFULL)
        def _(i=i):
            dma_wait(i)
            sc_issue(i)
        if i >= 2:
            @pl.when(base + i - 2 <= LASTFULL)
            def _(i=i):
                sc_wait(i - 2)
        if i + 2 < NB:
            @pl.when(base + i + 2 <= LASTFULL)
            def _(i=i):
                dma_issue(i + 2)
    for j in (NB - 2, NB - 1):
        @pl.when(base + j <= LASTFULL)
        def _(j=j):
            sc_wait(j)

    @pl.when(w == OWNER)
    def _():
        @pl.loop(NREST, BLK)
        def _(r):
            @pl.loop(0, D, step=16)
            def _(j):
                rows_v[0, r, pl.ds(j, 16)] = zz

        pltpu.sync_copy(x_hbm.at[pl.ds((LASTFULL + 1) * BLK, NREST)],
                        rows_v.at[0, pl.ds(0, NREST)])
        pltpu.sync_copy(rows_v.at[0], acc_sh.at[idx_v.at[OWNIDX]], add=True)

    plsc.subcore_barrier()
    pltpu.sync_copy(acc_sh.at[pl.ds(s * GROWS, GROWS)],
                    out_hbm.at[c, pl.ds(s * GROWS, GROWS)])


def _tc_pool_kernel(x_ref, ids_ref, out_ref, acc_ref):
    i = pl.program_id(0)

    @pl.when(i == 0)
    def _():
        acc_ref[...] = jnp.zeros_like(acc_ref)

    ids = ids_ref[0, 0, :]
    x = x_ref[...].astype(jnp.bfloat16)

    base = ids_ref[0, 0, 0]
    win_iota = jax.lax.broadcasted_iota(jnp.int32, (W, CHUNK), 0) + base
    onehot_w = (win_iota == ids[None, :]).astype(jnp.bfloat16)
    acc_ref[pl.ds(base, W), :] += jnp.dot(
        onehot_w, x, preferred_element_type=jnp.float32)

    @pl.when(ids[CHUNK - 1] >= base + W)
    def _():
        seg_iota = jax.lax.broadcasted_iota(jnp.int32, (G, CHUNK), 0)
        onehot_f = ((seg_iota == ids[None, :])
                    & (ids[None, :] >= base + W)).astype(jnp.bfloat16)
        acc_ref[pl.ds(0, G), :] += jnp.dot(
            onehot_f, x, preferred_element_type=jnp.float32)

    @pl.when(i == TSTEPS - 1)
    def _():
        out_ref[...] = acc_ref[pl.ds(0, G), :]


def _mlp_kernel(p_ref, t_ref, w1_ref, b1_ref, w2_ref, b2_ref, w3_ref,
                b3_ref, out_ref):
    g = p_ref[0] + p_ref[1] + t_ref[...]
    h = jnp.maximum(
        jnp.dot(g, w1_ref[...], preferred_element_type=jnp.float32)
        + b1_ref[...], 0.0)
    h = jnp.maximum(
        jnp.dot(h, w2_ref[...], preferred_element_type=jnp.float32)
        + b2_ref[...], 0.0)
    out_ref[...] = (
        jnp.dot(h, w3_ref[...], preferred_element_type=jnp.float32)
        + b3_ref[...])


@jax.jit
def kernel(atom_feat, batch, W1, b1, W2, b2, W3, b3):
    ids = batch.astype(jnp.int32)
    idsp = jnp.pad(
        jnp.pad(ids[T:], (0, NBLOCKS * BLK - N)).reshape(NW, NB, BLK),
        ((0, 0), (0, NBP - NB), (0, 0))).reshape(NW * NBP, BLK)

    mesh = plsc.VectorSubcoreMesh(core_axis_name="c", subcore_axis_name="s")
    sc_pool = partial(
        pl.kernel,
        mesh=mesh,
        out_type=jax.ShapeDtypeStruct((NC, G, D), jnp.float32),
        scratch_types=[
            pltpu.VMEM((NBUF, BLK, D), jnp.float32),
            pltpu.VMEM((NBP, BLK), jnp.int32),
            pltpu.VMEM_SHARED((G, D), jnp.float32),
            pltpu.SemaphoreType.DMA,
            pltpu.SemaphoreType.DMA,
            pltpu.SemaphoreType.DMA,
            pltpu.SemaphoreType.DMA,
            pltpu.SemaphoreType.DMA,
            pltpu.SemaphoreType.DMA,
        ],
    )(_sc_pool)
    partials = sc_pool(atom_feat, idsp)

    ids3 = ids[:T].reshape(TSTEPS, 1, CHUNK)
    tc_part = pl.pallas_call(
        _tc_pool_kernel,
        grid=(TSTEPS,),
        in_specs=[
            pl.BlockSpec((CHUNK, D), lambda i: (i, 0)),
            pl.BlockSpec((1, 1, CHUNK), lambda i: (i, 0, 0)),
        ],
        out_specs=pl.BlockSpec((G, D), lambda i: (0, 0)),
        out_shape=jax.ShapeDtypeStruct((G, D), jnp.float32),
        scratch_shapes=[pltpu.VMEM((G + W, D), jnp.float32)],
        compiler_params=pltpu.CompilerParams(
            dimension_semantics=("arbitrary",)),
    )(atom_feat, ids3)

    out = pl.pallas_call(
        _mlp_kernel,
        out_shape=jax.ShapeDtypeStruct((G, 1), jnp.float32),
    )(partials, tc_part, W1, b1.reshape(1, H1), W2, b2.reshape(1, H2),
      W3, b3.reshape(1, 1))
    return out


# speedup vs baseline: 1.0765x; 1.0037x over previous
import jax
import jax.numpy as jnp
from jax import lax
from jax.experimental import pallas as pl
from jax.experimental.pallas import tpu as pltpu
from jax.experimental.pallas import tpu_sc as plsc
from functools import partial

N = 100000
D = 128
G = 1024
H1 = 256
H2 = 128

NC = 2
NS = 16
NW = NC * NS
BLK = 128
NBP = 32
NBUF = 4
NBLOCKS = 800
LASTFULL = N // BLK - 1
NREST = N - (LASTFULL + 1) * BLK
GROWS = G // NS

T = 36864
TBLK = T // BLK
NB = (NBLOCKS - TBLK) // NW
OWNER = (LASTFULL + 1 - TBLK) // NB
OWNIDX = LASTFULL + 1 - (TBLK + OWNER * NB)
CHUNK = 2048
TSTEPS = T // CHUNK
W = 256


def _sc_pool(x_hbm, idsp_hbm, out_hbm, rows_v, idx_v, acc_sh,
             d0, d1, d2, d3, s0, s1):
    c = lax.axis_index("c")
    s = lax.axis_index("s")
    w = c * NS + s

    dsems = (d0, d1, d2, d3)
    ssems = (s0, s1)

    zz = jnp.zeros((16,), jnp.float32)

    @pl.loop(0, GROWS)
    def _(r):
        @pl.loop(0, D, step=16)
        def _(j):
            rows_v[0, r, pl.ds(j, 16)] = zz

    pltpu.sync_copy(rows_v.at[0, pl.ds(0, GROWS)],
                    acc_sh.at[pl.ds(s * GROWS, GROWS)])
    plsc.subcore_barrier()

    pltpu.sync_copy(idsp_hbm.at[pl.ds(w * NBP, NBP)], idx_v)

    base = TBLK + w * NB

    def dma_issue(j):
        pltpu.async_copy(x_hbm.at[pl.ds((base + j) * BLK, BLK)],
                         rows_v.at[j % NBUF], dsems[j % NBUF])

    def dma_wait(j):
        pltpu.make_async_copy(x_hbm.at[pl.ds(0, BLK)],
                              rows_v.at[j % NBUF], dsems[j % NBUF]).wait()

    def sc_issue(j):
        pltpu.async_copy(rows_v.at[j % NBUF],
                         acc_sh.at[idx_v.at[j]], ssems[j % 2], add=True)

    def sc_wait(j):
        pltpu.make_async_copy(rows_v.at[j % NBUF],
                              acc_sh.at[idx_v.at[j]], ssems[j % 2]).wait()

    @pl.when(base <= LASTFULL)
    def _():
        dma_issue(0)

    @pl.when(base + 1 <= LASTFULL)
    def _():
        dma_issue(1)

    for i in range(NB):
        @pl.when(base + i <= LASTFULL)
        def _(i=i):
            dma_wait(i)
            sc_issue(i)
        if i >= 2:
            @pl.when(base + i - 2 <= LASTFULL)
            def _(i=i):
                sc_wait(i - 2)
        if i + 2 < NB:
            @pl.when(base + i + 2 <= LASTFULL)
            def _(i=i):
                dma_issue(i + 2)
    for j in (NB - 2, NB - 1):
        @pl.when(base + j <= LASTFULL)
        def _(j=j):
            sc_wait(j)

    @pl.when(w == OWNER)
    def _():
        @pl.loop(NREST, BLK)
        def _(r):
            @pl.loop(0, D, step=16)
            def _(j):
                rows_v[0, r, pl.ds(j, 16)] = zz

        pltpu.sync_copy(x_hbm.at[pl.ds((LASTFULL + 1) * BLK, NREST)],
                        rows_v.at[0, pl.ds(0, NREST)])
        pltpu.sync_copy(rows_v.at[0], acc_sh.at[idx_v.at[OWNIDX]], add=True)

    plsc.subcore_barrier()
    pltpu.sync_copy(acc_sh.at[pl.ds(s * GROWS, GROWS)],
                    out_hbm.at[c, pl.ds(s * GROWS, GROWS)])


def _tc_pool_kernel(x_ref, ids_ref, out_ref, acc_ref):
    i = pl.program_id(0)

    @pl.when(i == 0)
    def _():
        acc_ref[...] = jnp.zeros_like(acc_ref)

    ids = ids_ref[0, 0, :]
    x = x_ref[...].astype(jnp.bfloat16)

    base = ids_ref[0, 0, 0]
    win_iota = jax.lax.broadcasted_iota(jnp.int32, (W, CHUNK), 0) + base
    onehot_w = (win_iota == ids[None, :]).astype(jnp.bfloat16)
    acc_ref[pl.ds(base, W), :] += jnp.dot(
        onehot_w, x, preferred_element_type=jnp.float32)

    @pl.when(ids[CHUNK - 1] >= base + W)
    def _():
        seg_iota = jax.lax.broadcasted_iota(jnp.int32, (G, CHUNK), 0)
        onehot_f = ((seg_iota == ids[None, :])
                    & (ids[None, :] >= base + W)).astype(jnp.bfloat16)
        acc_ref[pl.ds(0, G), :] += jnp.dot(
            onehot_f, x, preferred_element_type=jnp.float32)

    @pl.when(i == TSTEPS - 1)
    def _():
        out_ref[...] = acc_ref[pl.ds(0, G), :]


def _mlp_kernel(p_ref, t_ref, w1_ref, b1_ref, w2_ref, b2_ref, w3_ref,
                b3_ref, out_ref):
    g = p_ref[0] + p_ref[1] + t_ref[...]
    h = jnp.maximum(
        jnp.dot(g, w1_ref[...], preferred_element_type=jnp.float32)
        + b1_ref[...], 0.0)
    h = jnp.maximum(
        jnp.dot(h, w2_ref[...], preferred_element_type=jnp.float32)
        + b2_ref[...], 0.0)
    out_ref[...] = (
        jnp.dot(h, w3_ref[...], preferred_element_type=jnp.float32)
        + b3_ref[...])


@jax.jit
def kernel(atom_feat, batch, W1, b1, W2, b2, W3, b3):
    ids = batch.astype(jnp.int32)
    idsp = jnp.pad(
        jnp.pad(ids[T:], (0, NBLOCKS * BLK - N)).reshape(NW, NB, BLK),
        ((0, 0), (0, NBP - NB), (0, 0))).reshape(NW * NBP, BLK)

    mesh = plsc.VectorSubcoreMesh(core_axis_name="c", subcore_axis_name="s")
    sc_pool = partial(
        pl.kernel,
        mesh=mesh,
        out_type=jax.ShapeDtypeStruct((NC, G, D), jnp.float32),
        scratch_types=[
            pltpu.VMEM((NBUF, BLK, D), jnp.float32),
            pltpu.VMEM((NBP, BLK), jnp.int32),
            pltpu.VMEM_SHARED((G, D), jnp.float32),
            pltpu.SemaphoreType.DMA,
            pltpu.SemaphoreType.DMA,
            pltpu.SemaphoreType.DMA,
            pltpu.SemaphoreType.DMA,
            pltpu.SemaphoreType.DMA,
            pltpu.SemaphoreType.DMA,
        ],
    )(_sc_pool)
    partials = sc_pool(atom_feat, idsp)

    ids3 = ids[:T].reshape(TSTEPS, 1, CHUNK)
    tc_part = pl.pallas_call(
        _tc_pool_kernel,
        grid=(TSTEPS,),
        in_specs=[
            pl.BlockSpec((CHUNK, D), lambda i: (i, 0)),
            pl.BlockSpec((1, 1, CHUNK), lambda i: (i, 0, 0)),
        ],
        out_specs=pl.BlockSpec((G, D), lambda i: (0, 0)),
        out_shape=jax.ShapeDtypeStruct((G, D), jnp.float32),
        scratch_shapes=[pltpu.VMEM((G + W, D), jnp.float32)],
        compiler_params=pltpu.CompilerParams(
            dimension_semantics=("arbitrary",)),
    )(atom_feat, ids3)

    out = pl.pallas_call(
        _mlp_kernel,
        out_shape=jax.ShapeDtypeStruct((G, 1), jnp.float32),
    )(partials, tc_part, W1, b1.reshape(1, H1), W2, b2.reshape(1, H2),
      W3, b3.reshape(1, 1))
    return out
